# baseline (speedup 1.0000x reference)
import jax
import jax.numpy as jnp
from jax.experimental import pallas as pl
from jax.experimental.pallas import tpu as pltpu

N_QUERY = 4096
N_TRAIN = 16384
N_FEAT = 128
N_OUT = 16
TOP_K = 16
Q_BLOCK = 256
N_BLOCKS = N_QUERY // Q_BLOCK


def _bitonic_merge(lst):
    n = len(lst)
    if n == 1:
        return lst
    h = n // 2
    hi = [jnp.maximum(lst[i], lst[i + h]) for i in range(h)]
    lo = [jnp.minimum(lst[i], lst[i + h]) for i in range(h)]
    return _bitonic_merge(hi) + _bitonic_merge(lo)


_SORT16_PAIRS = [
    (0, 1), (2, 3), (4, 5), (6, 7), (8, 9), (10, 11), (12, 13), (14, 15),
    (0, 2), (1, 3), (4, 6), (5, 7), (8, 10), (9, 11), (12, 14), (13, 15),
    (0, 4), (1, 5), (2, 6), (3, 7), (8, 12), (9, 13), (10, 14), (11, 15),
    (0, 8), (1, 9), (2, 10), (3, 11), (4, 12), (5, 13), (6, 14), (7, 15),
    (5, 10), (6, 9), (3, 12), (13, 14), (7, 11), (1, 2), (4, 8),
    (1, 4), (7, 13), (2, 8), (11, 14), (5, 6), (9, 10),
    (2, 4), (11, 13), (3, 8), (7, 12),
    (6, 8), (10, 12), (3, 5), (7, 9),
    (3, 4), (5, 6), (7, 8), (9, 10), (11, 12),
    (6, 7), (8, 9),
]


def _top_k_threshold(d):
    w = d.shape[1] // TOP_K
    lst = [d[:, i * w:(i + 1) * w] for i in range(TOP_K)]
    for i, j in _SORT16_PAIRS:
        hi = jnp.maximum(lst[i], lst[j])
        lo = jnp.minimum(lst[i], lst[j])
        lst[i], lst[j] = hi, lo
    while lst[0].shape[1] > 128:
        half = lst[0].shape[1] // 2
        a = [t[:, :half] for t in lst]
        b = [t[:, half:] for t in lst]
        rev = b[::-1]
        hi = [jnp.maximum(a[i], rev[i]) for i in range(TOP_K)]
        lst = _bitonic_merge(hi)
    lst = [jnp.swapaxes(t, 0, 1) for t in lst]
    while True:
        half = lst[0].shape[0] // 2
        a = [t[:half] for t in lst]
        b = [t[half:] for t in lst]
        rev = b[::-1]
        hi = [jnp.maximum(a[i], rev[i]) for i in range(TOP_K)]
        if half == 1:
            while len(hi) > 1:
                hi = [jnp.minimum(hi[j], hi[j + len(hi) // 2])
                      for j in range(len(hi) // 2)]
            return jnp.swapaxes(hi[0], 0, 1)
        lst = _bitonic_merge(hi)


def _idw_block_kernel(x_ref, tst_ref, ty_ref, w_ref, out_ref, dist_ref,
                      y2_ref):
    s2 = jnp.exp(-2.0 * w_ref[0, :])

    @pl.when(pl.program_id(0) == 0)
    def _():
        tst = tst_ref[...]
        y2_ref[...] = jnp.sum(tst * tst * s2[:, None], axis=0, keepdims=True)

    x = x_ref[...]
    xs = x * s2[None, :]
    x2 = jnp.sum(x * xs, axis=1, keepdims=True)
    cross = jnp.dot(-2.0 * xs, tst_ref[...],
                    preferred_element_type=jnp.float32)
    sq = jnp.maximum(x2 + y2_ref[...] + cross, 0.0)
    dist_ref[...] = jax.lax.rsqrt(sq + 1e-6)

    d = dist_ref[...]
    thr = _top_k_threshold(d)
    wts = jnp.where(d >= thr, d, 0.0)
    num = jnp.dot(wts, ty_ref[...], preferred_element_type=jnp.float32)
    out_ref[...] = num[:, :N_OUT] / num[:, N_OUT:N_OUT + 1]


def kernel(x, train_x, train_y, w):
    tst = train_x.T
    w2d = w.reshape(1, N_FEAT)
    ty_aug = jnp.concatenate(
        [train_y, jnp.ones((N_TRAIN, 1), jnp.float32)], axis=1)
    return pl.pallas_call(
        _idw_block_kernel,
        grid=(N_BLOCKS,),
        in_specs=[
            pl.BlockSpec((Q_BLOCK, N_FEAT), lambda i: (i, 0)),
            pl.BlockSpec((N_FEAT, N_TRAIN), lambda i: (0, 0)),
            pl.BlockSpec((N_TRAIN, N_OUT + 1), lambda i: (0, 0)),
            pl.BlockSpec((1, N_FEAT), lambda i: (0, 0)),
        ],
        out_specs=pl.BlockSpec((Q_BLOCK, N_OUT), lambda i: (i, 0)),
        out_shape=jax.ShapeDtypeStruct((N_QUERY, N_OUT), jnp.float32),
        scratch_shapes=[
            pltpu.VMEM((Q_BLOCK, N_TRAIN), jnp.float32),
            pltpu.VMEM((1, N_TRAIN), jnp.float32),
        ],
        compiler_params=pltpu.CompilerParams(
            dimension_semantics=("arbitrary",)),
    )(x, tst, ty_aug, w2d)


# speedup vs baseline: 1.0592x; 1.0592x over previous
import jax
import jax.numpy as jnp
from jax.experimental import pallas as pl
from jax.experimental.pallas import tpu as pltpu

N_QUERY = 4096
N_TRAIN = 16384
N_FEAT = 128
N_OUT = 16
TOP_K = 16
Q_BLOCK = 256
N_BLOCKS = N_QUERY // Q_BLOCK


def _bitonic_merge(lst):
    n = len(lst)
    if n == 1:
        return lst
    h = n // 2
    hi = [jnp.maximum(lst[i], lst[i + h]) for i in range(h)]
    lo = [jnp.minimum(lst[i], lst[i + h]) for i in range(h)]
    return _bitonic_merge(hi) + _bitonic_merge(lo)


def _batcher_pairs(n):
    pairs = []

    def merge(lo, length, r):
        m = r * 2
        if m < length:
            merge(lo, length, m)
            merge(lo + r, length, m)
            for i in range(lo + r, lo + length - r, m):
                pairs.append((i, i + r))
        else:
            pairs.append((lo, lo + r))

    def sort(lo, length):
        if length > 1:
            m = length // 2
            sort(lo, m)
            sort(lo + m, m)
            merge(lo, length, 1)

    sort(0, n)
    return pairs


_SORT16_PAIRS = _batcher_pairs(TOP_K)


def _top_k_threshold(d):
    w = d.shape[1] // TOP_K
    lst = [d[:, i * w:(i + 1) * w] for i in range(TOP_K)]
    for i, j in _SORT16_PAIRS:
        hi = jnp.maximum(lst[i], lst[j])
        lo = jnp.minimum(lst[i], lst[j])
        lst[i], lst[j] = hi, lo
    while lst[0].shape[1] > 128:
        half = lst[0].shape[1] // 2
        a = [t[:, :half] for t in lst]
        b = [t[:, half:] for t in lst]
        rev = b[::-1]
        hi = [jnp.maximum(a[i], rev[i]) for i in range(TOP_K)]
        lst = _bitonic_merge(hi)
    lst = [jnp.swapaxes(t, 0, 1) for t in lst]
    while True:
        half = lst[0].shape[0] // 2
        a = [t[:half] for t in lst]
        b = [t[half:] for t in lst]
        rev = b[::-1]
        hi = [jnp.maximum(a[i], rev[i]) for i in range(TOP_K)]
        if half == 1:
            while len(hi) > 1:
                hi = [jnp.minimum(hi[j], hi[j + len(hi) // 2])
                      for j in range(len(hi) // 2)]
            return jnp.swapaxes(hi[0], 0, 1)
        lst = _bitonic_merge(hi)


def _idw_block_kernel(x_ref, tst_ref, ty_ref, w_ref, out_ref, dist_ref,
                      y2_ref):
    s2 = jnp.exp(-2.0 * w_ref[0, :])

    @pl.when(pl.program_id(0) == 0)
    def _():
        tst = tst_ref[...]
        y2_ref[...] = jnp.sum(tst * tst * s2[:, None], axis=0, keepdims=True)

    x = x_ref[...]
    xs = x * s2[None, :]
    x2 = jnp.sum(x * xs, axis=1, keepdims=True)
    cross = jnp.dot(-2.0 * xs, tst_ref[...],
                    preferred_element_type=jnp.float32)
    sq = jnp.maximum(x2 + y2_ref[...] + cross, 0.0)
    dist_ref[...] = jax.lax.rsqrt(sq + 1e-6)

    d = dist_ref[...]
    thr = _top_k_threshold(d)
    wts = jnp.where(d >= thr, d, 0.0)
    num = jnp.dot(wts, ty_ref[...], preferred_element_type=jnp.float32)
    out_ref[...] = num[:, :N_OUT] / num[:, N_OUT:N_OUT + 1]


def kernel(x, train_x, train_y, w):
    tst = train_x.T
    w2d = w.reshape(1, N_FEAT)
    ty_aug = jnp.concatenate(
        [train_y, jnp.ones((N_TRAIN, 1), jnp.float32)], axis=1)
    return pl.pallas_call(
        _idw_block_kernel,
        grid=(N_BLOCKS,),
        in_specs=[
            pl.BlockSpec((Q_BLOCK, N_FEAT), lambda i: (i, 0)),
            pl.BlockSpec((N_FEAT, N_TRAIN), lambda i: (0, 0)),
            pl.BlockSpec((N_TRAIN, N_OUT + 1), lambda i: (0, 0)),
            pl.BlockSpec((1, N_FEAT), lambda i: (0, 0)),
        ],
        out_specs=pl.BlockSpec((Q_BLOCK, N_OUT), lambda i: (i, 0)),
        out_shape=jax.ShapeDtypeStruct((N_QUERY, N_OUT), jnp.float32),
        scratch_shapes=[
            pltpu.VMEM((Q_BLOCK, N_TRAIN), jnp.float32),
            pltpu.VMEM((1, N_TRAIN), jnp.float32),
        ],
        compiler_params=pltpu.CompilerParams(
            dimension_semantics=("arbitrary",)),
    )(x, tst, ty_aug, w2d)
